# baseline (speedup 1.0000x reference)
import functools

import jax
import jax.numpy as jnp
from jax import lax
from jax.experimental import pallas as pl
from jax.experimental.pallas import tpu as pltpu
from jax.experimental.pallas import tpu_sc as plsc

NC = 2
NS = 16
LANES = 16
CHUNK = 128


def _sc_degree(dst_deg, ones_in, zrows, n, nacc, nchw, width):
    rowz = nacc // NS
    mesh = plsc.VectorSubcoreMesh(core_axis_name="c", subcore_axis_name="s")

    @functools.partial(
        pl.kernel,
        mesh=mesh,
        out_type=jax.ShapeDtypeStruct((NC, nacc, width), jnp.float32),
        scratch_types=[
            pltpu.VMEM((nchw, CHUNK), jnp.int32),
            pltpu.VMEM((CHUNK, width), jnp.float32),
            pltpu.VMEM_SHARED((nacc, width), jnp.float32),
        ],
    )
    def degk(dst_hbm, ones_hbm, zrows_hbm, out_hbm, dstv, onesv, acc):
        c = lax.axis_index("c")
        s = lax.axis_index("s")
        w = s * NC + c
        pltpu.sync_copy(zrows_hbm, acc.at[pl.ds(s * rowz, rowz)])
        pltpu.sync_copy(dst_hbm.at[w], dstv)
        pltpu.sync_copy(ones_hbm, onesv)
        plsc.subcore_barrier()

        def body(j, carry):
            pltpu.sync_copy(onesv, acc.at[dstv.at[j]], add=True)
            return carry

        lax.fori_loop(0, nchw, body, 0)
        plsc.subcore_barrier()
        pltpu.sync_copy(acc.at[pl.ds(s * rowz, rowz)],
                        out_hbm.at[c, pl.ds(s * rowz, rowz)])

    return degk(dst_deg, ones_in, zrows)


def _sc_aggregate(z_flat, src_idx, dst_main, zrows, n, half, nacc, nch):
    rowz = nacc // NS
    mesh = plsc.VectorSubcoreMesh(core_axis_name="c", subcore_axis_name="s")

    nch2 = nch // 2

    @functools.partial(
        pl.kernel,
        mesh=mesh,
        out_type=jax.ShapeDtypeStruct((NC * nacc, half), jnp.float32),
        scratch_types=[
            pltpu.VMEM((nch2, CHUNK), jnp.int32),
            pltpu.VMEM((nch2, CHUNK), jnp.int32),
            pltpu.VMEM((CHUNK, half), jnp.float32),
            pltpu.VMEM((CHUNK, half), jnp.float32),
            pltpu.VMEM_SHARED((nacc, half), jnp.float32),
            pltpu.SemaphoreType.DMA,
            pltpu.SemaphoreType.DMA,
            pltpu.SemaphoreType.DMA,
            pltpu.SemaphoreType.DMA,
        ],
    )
    def aggk(z_hbm, src_hbm, dst_hbm, zrows_hbm, out_hbm,
             srcv, dstv, gbuf0, gbuf1, acc, g0, g1, s0, s1):
        c = lax.axis_index("c")
        s = lax.axis_index("s")
        pltpu.sync_copy(zrows_hbm, acc.at[pl.ds(s * rowz, rowz)])
        plsc.subcore_barrier()

        for p in range(2):
            pltpu.sync_copy(src_hbm.at[c, s, pl.ds(p * nch2, nch2)], srcv)
            pltpu.sync_copy(dst_hbm.at[s, pl.ds(p * nch2, nch2)], dstv)
            pltpu.async_copy(z_hbm.at[srcv.at[0]], gbuf0, g0)
            pltpu.async_copy(z_hbm.at[srcv.at[1]], gbuf1, g1)

            def body(i, carry):
                j = 2 * i
                jn = jnp.minimum(j + 2, nch2 - 1)
                jn1 = jnp.minimum(j + 3, nch2 - 1)
                pltpu.make_async_copy(z_hbm.at[srcv.at[j]], gbuf0, g0).wait()
                pltpu.async_copy(gbuf0, acc.at[dstv.at[j]], s0, add=True)
                pltpu.make_async_copy(z_hbm.at[srcv.at[j + 1]], gbuf1, g1).wait()
                pltpu.async_copy(gbuf1, acc.at[dstv.at[j + 1]], s1, add=True)
                pltpu.make_async_copy(gbuf0, acc.at[dstv.at[j]], s0).wait()
                pltpu.async_copy(z_hbm.at[srcv.at[jn]], gbuf0, g0)
                pltpu.make_async_copy(gbuf1, acc.at[dstv.at[j + 1]], s1).wait()
                pltpu.async_copy(z_hbm.at[srcv.at[jn1]], gbuf1, g1)
                return carry

            lax.fori_loop(0, nch2 // 2, body, 0)
            pltpu.make_async_copy(z_hbm.at[srcv.at[0]], gbuf0, g0).wait()
            pltpu.make_async_copy(z_hbm.at[srcv.at[1]], gbuf1, g1).wait()
        plsc.subcore_barrier()
        pltpu.sync_copy(acc.at[pl.ds(s * rowz, rowz)],
                        out_hbm.at[pl.ds(c * nacc + s * rowz, rowz)])

    return aggk(z_flat, src_idx, dst_main, zrows)


RBLK = 2000


def _mm(a, b):
    return jnp.dot(a, b, preferred_element_type=jnp.float32,
                   precision=lax.Precision.HIGHEST)


def _tc_first(x, w_split, degp, n, half):
    def body(x_ref, w_ref, degp_ref, z_ref, dinv_ref):
        deg = degp_ref[0, :, 0:1] + degp_ref[1, :, 0:1] + 1.0
        dinv = lax.rsqrt(deg)
        dinv_ref[...] = dinv
        xv = x_ref[...]
        for c in range(NC):
            y = _mm(xv[:, :half], w_ref[0, :, c * half:(c + 1) * half])
            y += _mm(xv[:, half:], w_ref[1, :, c * half:(c + 1) * half])
            z_ref[c] = dinv * y

    return pl.pallas_call(
        body,
        grid=(n // RBLK,),
        in_specs=[
            pl.BlockSpec((RBLK, 2 * half), lambda i: (i, 0)),
            pl.BlockSpec((NC, half, 2 * half), lambda i: (0, 0, 0)),
            pl.BlockSpec((NC, RBLK, half), lambda i: (0, i, 0)),
        ],
        out_specs=[
            pl.BlockSpec((NC, RBLK, half), lambda i: (0, i, 0)),
            pl.BlockSpec((RBLK, 1), lambda i: (i, 0)),
        ],
        out_shape=[jax.ShapeDtypeStruct((NC, n, half), jnp.float32),
                   jax.ShapeDtypeStruct((n, 1), jnp.float32)],
    )(x, w_split, degp)


def _tc_mid(aggp, zp, dinv, b_split, w_split, n, half):
    def body(aggp_ref, zp_ref, dinv_ref, b_ref, w_ref, zout_ref):
        dinv = dinv_ref[...]
        h0 = jnp.maximum(dinv * (aggp_ref[0] + zp_ref[0]) + b_ref[0], 0.0)
        h1 = jnp.maximum(dinv * (aggp_ref[1] + zp_ref[1]) + b_ref[1], 0.0)
        for c in range(NC):
            y = _mm(h0, w_ref[0, :, c * half:(c + 1) * half])
            y += _mm(h1, w_ref[1, :, c * half:(c + 1) * half])
            zout_ref[c] = dinv * y

    return pl.pallas_call(
        body,
        grid=(n // RBLK,),
        in_specs=[
            pl.BlockSpec((NC, RBLK, half), lambda i: (0, i, 0)),
            pl.BlockSpec((NC, RBLK, half), lambda i: (0, i, 0)),
            pl.BlockSpec((RBLK, 1), lambda i: (i, 0)),
            pl.BlockSpec((NC, 1, half), lambda i: (0, 0, 0)),
            pl.BlockSpec((NC, half, 2 * half), lambda i: (0, 0, 0)),
        ],
        out_specs=pl.BlockSpec((NC, RBLK, half), lambda i: (0, i, 0)),
        out_shape=jax.ShapeDtypeStruct((NC, n, half), jnp.float32),
    )(aggp, zp, dinv, b_split, w_split)


def _tc_final(aggp, zp, dinv, b_split, wfc_split, bfc, n, half, out_d):
    def body(aggp_ref, zp_ref, dinv_ref, b_ref, wfc_ref, bfc_ref,
             emb_ref, logits_ref):
        i = pl.program_id(0)
        dinv = dinv_ref[...]
        e0 = jnp.maximum(dinv * (aggp_ref[0] + zp_ref[0]) + b_ref[0], 0.0)
        e1 = jnp.maximum(dinv * (aggp_ref[1] + zp_ref[1]) + b_ref[1], 0.0)
        emb_ref[:, 0:half] = e0
        emb_ref[:, half:2 * half] = e1
        inv_n = 1.0 / float(n)
        p0 = jnp.sum(e0, axis=0, keepdims=True) * inv_n
        p1 = jnp.sum(e1, axis=0, keepdims=True) * inv_n
        part = _mm(p0, wfc_ref[0]) + _mm(p1, wfc_ref[1])

        @pl.when(i == 0)
        def _():
            logits_ref[...] = bfc_ref[...]

        logits_ref[...] += part

    return pl.pallas_call(
        body,
        grid=(n // RBLK,),
        in_specs=[
            pl.BlockSpec((NC, RBLK, half), lambda i: (0, i, 0)),
            pl.BlockSpec((NC, RBLK, half), lambda i: (0, i, 0)),
            pl.BlockSpec((RBLK, 1), lambda i: (i, 0)),
            pl.BlockSpec((NC, 1, half), lambda i: (0, 0, 0)),
            pl.BlockSpec((NC, half, out_d), lambda i: (0, 0, 0)),
            pl.BlockSpec((1, out_d), lambda i: (0, 0)),
        ],
        out_specs=[
            pl.BlockSpec((RBLK, 2 * half), lambda i: (i, 0)),
            pl.BlockSpec((1, out_d), lambda i: (0, 0)),
        ],
        out_shape=[jax.ShapeDtypeStruct((n, 2 * half), jnp.float32),
                   jax.ShapeDtypeStruct((1, out_d), jnp.float32)],
    )(aggp, zp, dinv, b_split, wfc_split, bfc)


def kernel(x, edge_index, batch, W1, b1, W2, b2, W3, b3, Wfc, bfc):
    n, d = x.shape
    e = edge_index.shape[1]
    out_d = Wfc.shape[1]
    half = d // 2

    grain = 2 * NC * NS * CHUNK
    e2 = ((e + grain - 1) // grain) * grain
    nch = e2 // (NS * CHUNK)
    nchw = e2 // (NC * NS * CHUNK)
    nacc = ((n + 1 + NS * 8 - 1) // (NS * 8)) * (NS * 8)
    rowz = nacc // NS

    src = edge_index[0]
    dst = edge_index[1]
    pad = e2 - e
    src_p = jnp.concatenate([src, jnp.zeros((pad,), jnp.int32)])
    dst_p = jnp.concatenate([dst, jnp.full((pad,), n, jnp.int32)])
    src_idx = jnp.stack([src_p, src_p + n]).reshape(NC, NS, nch, CHUNK)
    dst_main = dst_p.reshape(NS, nch, CHUNK)
    dst_deg = dst_p.reshape(NC * NS, nchw, CHUNK)
    ones_in = jnp.ones((CHUNK, half), jnp.float32)
    zrows128 = jnp.zeros((rowz, half), jnp.float32)

    degp = _sc_degree(dst_deg, ones_in, zrows128, n, nacc, nchw, half)

    w1s = W1.reshape(NC, half, d)
    w2s = W2.reshape(NC, half, d)
    w3s = W3.reshape(NC, half, d)
    wfcs = Wfc.reshape(NC, half, out_d)
    b1s = b1.reshape(NC, 1, half)
    b2s = b2.reshape(NC, 1, half)
    b3s = b3.reshape(NC, 1, half)
    bfc2 = bfc.reshape(1, out_d)

    z1, dinv = _tc_first(x, w1s, degp, n, half)
    agg1 = _sc_aggregate(z1.reshape(NC * n, half), src_idx, dst_main,
                         zrows128, n, half, nacc, nch
                         ).reshape(NC, nacc, half)[:, :n, :]
    z2 = _tc_mid(agg1, z1, dinv, b1s, w2s, n, half)
    agg2 = _sc_aggregate(z2.reshape(NC * n, half), src_idx, dst_main,
                         zrows128, n, half, nacc, nch
                         ).reshape(NC, nacc, half)[:, :n, :]
    z3 = _tc_mid(agg2, z2, dinv, b2s, w3s, n, half)
    agg3 = _sc_aggregate(z3.reshape(NC * n, half), src_idx, dst_main,
                         zrows128, n, half, nacc, nch
                         ).reshape(NC, nacc, half)[:, :n, :]
    emb, logits = _tc_final(agg3, z3, dinv, b3s, wfcs, bfc2, n, half, out_d)
    return (logits, emb)


# speedup vs baseline: 1.0667x; 1.0667x over previous
import functools

import jax
import jax.numpy as jnp
from jax import lax
from jax.experimental import pallas as pl
from jax.experimental.pallas import tpu as pltpu
from jax.experimental.pallas import tpu_sc as plsc

NC = 2
NS = 16
LANES = 16
CHUNK = 128


def _sc_degree(dst_deg, ones_in, zrows, n, nacc, nchw, width):
    rowz = nacc // NS
    mesh = plsc.VectorSubcoreMesh(core_axis_name="c", subcore_axis_name="s")

    @functools.partial(
        pl.kernel,
        mesh=mesh,
        out_type=jax.ShapeDtypeStruct((NC, nacc, width), jnp.float32),
        scratch_types=[
            pltpu.VMEM((nchw, CHUNK), jnp.int32),
            pltpu.VMEM((CHUNK, width), jnp.float32),
            pltpu.VMEM_SHARED((nacc, width), jnp.float32),
        ],
    )
    def degk(dst_hbm, ones_hbm, zrows_hbm, out_hbm, dstv, onesv, acc):
        c = lax.axis_index("c")
        s = lax.axis_index("s")
        w = s * NC + c
        pltpu.sync_copy(zrows_hbm, acc.at[pl.ds(s * rowz, rowz)])
        pltpu.sync_copy(dst_hbm.at[w], dstv)
        pltpu.sync_copy(ones_hbm, onesv)
        plsc.subcore_barrier()

        def body(j, carry):
            pltpu.sync_copy(onesv, acc.at[dstv.at[j]], add=True)
            return carry

        lax.fori_loop(0, nchw, body, 0)
        plsc.subcore_barrier()
        pltpu.sync_copy(acc.at[pl.ds(s * rowz, rowz)],
                        out_hbm.at[c, pl.ds(s * rowz, rowz)])

    return degk(dst_deg, ones_in, zrows)


def _sc_aggregate(z_flat, src_idx, dst_main, zrows, n, half, nacc, nch):
    rowz = nacc // NS
    mesh = plsc.VectorSubcoreMesh(core_axis_name="c", subcore_axis_name="s")

    nch2 = nch // 2

    @functools.partial(
        pl.kernel,
        mesh=mesh,
        out_type=jax.ShapeDtypeStruct((NC * nacc, half), jnp.float32),
        scratch_types=[
            pltpu.VMEM((nch2, CHUNK), jnp.int32),
            pltpu.VMEM((nch2, CHUNK), jnp.int32),
            pltpu.VMEM((CHUNK, half), jnp.float32),
            pltpu.VMEM((CHUNK, half), jnp.float32),
            pltpu.VMEM_SHARED((nacc, half), jnp.float32),
            pltpu.SemaphoreType.DMA,
            pltpu.SemaphoreType.DMA,
            pltpu.SemaphoreType.DMA,
            pltpu.SemaphoreType.DMA,
        ],
    )
    def aggk(z_hbm, src_hbm, dst_hbm, zrows_hbm, out_hbm,
             srcv, dstv, gbuf0, gbuf1, acc, g0, g1, s0, s1):
        c = lax.axis_index("c")
        s = lax.axis_index("s")
        pltpu.sync_copy(zrows_hbm, acc.at[pl.ds(s * rowz, rowz)])
        plsc.subcore_barrier()

        for p in range(2):
            pltpu.sync_copy(src_hbm.at[c, s, pl.ds(p * nch2, nch2)], srcv)
            pltpu.sync_copy(dst_hbm.at[s, pl.ds(p * nch2, nch2)], dstv)
            hc = CHUNK // 2

            def gather(j, buf, semA, semB):
                pltpu.async_copy(z_hbm.at[srcv.at[j, pl.ds(0, hc)]],
                                 buf.at[pl.ds(0, hc)], semA)
                pltpu.async_copy(z_hbm.at[srcv.at[j, pl.ds(hc, hc)]],
                                 buf.at[pl.ds(hc, hc)], semB)

            def gwait(buf, semA, semB):
                pltpu.make_async_copy(z_hbm.at[srcv.at[0, pl.ds(0, hc)]],
                                      buf.at[pl.ds(0, hc)], semA).wait()
                pltpu.make_async_copy(z_hbm.at[srcv.at[0, pl.ds(hc, hc)]],
                                      buf.at[pl.ds(hc, hc)], semB).wait()

            gather(0, gbuf0, g0, s0)

            def body(i, carry):
                j = 2 * i
                gather(j + 1, gbuf1, g1, s1)
                gwait(gbuf0, g0, s0)
                pltpu.sync_copy(gbuf0, acc.at[dstv.at[j]], add=True)
                jn = jnp.minimum(j + 2, nch2 - 1)
                gather(jn, gbuf0, g0, s0)
                gwait(gbuf1, g1, s1)
                pltpu.sync_copy(gbuf1, acc.at[dstv.at[j + 1]], add=True)
                return carry

            lax.fori_loop(0, nch2 // 2, body, 0)
            gwait(gbuf0, g0, s0)
        plsc.subcore_barrier()
        pltpu.sync_copy(acc.at[pl.ds(s * rowz, rowz)],
                        out_hbm.at[pl.ds(c * nacc + s * rowz, rowz)])

    return aggk(z_flat, src_idx, dst_main, zrows)


RBLK = 2000


def _mm(a, b):
    return jnp.dot(a, b, preferred_element_type=jnp.float32,
                   precision=lax.Precision.HIGHEST)


def _tc_first(x, w_split, degp, n, half):
    def body(x_ref, w_ref, degp_ref, z_ref, dinv_ref):
        deg = degp_ref[0, :, 0:1] + degp_ref[1, :, 0:1] + 1.0
        dinv = lax.rsqrt(deg)
        dinv_ref[...] = dinv
        xv = x_ref[...]
        for c in range(NC):
            y = _mm(xv[:, :half], w_ref[0, :, c * half:(c + 1) * half])
            y += _mm(xv[:, half:], w_ref[1, :, c * half:(c + 1) * half])
            z_ref[c] = dinv * y

    return pl.pallas_call(
        body,
        grid=(n // RBLK,),
        in_specs=[
            pl.BlockSpec((RBLK, 2 * half), lambda i: (i, 0)),
            pl.BlockSpec((NC, half, 2 * half), lambda i: (0, 0, 0)),
            pl.BlockSpec((NC, RBLK, half), lambda i: (0, i, 0)),
        ],
        out_specs=[
            pl.BlockSpec((NC, RBLK, half), lambda i: (0, i, 0)),
            pl.BlockSpec((RBLK, 1), lambda i: (i, 0)),
        ],
        out_shape=[jax.ShapeDtypeStruct((NC, n, half), jnp.float32),
                   jax.ShapeDtypeStruct((n, 1), jnp.float32)],
    )(x, w_split, degp)


def _tc_mid(aggp, zp, dinv, b_split, w_split, n, half):
    def body(aggp_ref, zp_ref, dinv_ref, b_ref, w_ref, zout_ref):
        dinv = dinv_ref[...]
        h0 = jnp.maximum(dinv * (aggp_ref[0] + zp_ref[0]) + b_ref[0], 0.0)
        h1 = jnp.maximum(dinv * (aggp_ref[1] + zp_ref[1]) + b_ref[1], 0.0)
        for c in range(NC):
            y = _mm(h0, w_ref[0, :, c * half:(c + 1) * half])
            y += _mm(h1, w_ref[1, :, c * half:(c + 1) * half])
            zout_ref[c] = dinv * y

    return pl.pallas_call(
        body,
        grid=(n // RBLK,),
        in_specs=[
            pl.BlockSpec((NC, RBLK, half), lambda i: (0, i, 0)),
            pl.BlockSpec((NC, RBLK, half), lambda i: (0, i, 0)),
            pl.BlockSpec((RBLK, 1), lambda i: (i, 0)),
            pl.BlockSpec((NC, 1, half), lambda i: (0, 0, 0)),
            pl.BlockSpec((NC, half, 2 * half), lambda i: (0, 0, 0)),
        ],
        out_specs=pl.BlockSpec((NC, RBLK, half), lambda i: (0, i, 0)),
        out_shape=jax.ShapeDtypeStruct((NC, n, half), jnp.float32),
    )(aggp, zp, dinv, b_split, w_split)


def _tc_final(aggp, zp, dinv, b_split, wfc_split, bfc, n, half, out_d):
    def body(aggp_ref, zp_ref, dinv_ref, b_ref, wfc_ref, bfc_ref,
             emb_ref, logits_ref):
        i = pl.program_id(0)
        dinv = dinv_ref[...]
        e0 = jnp.maximum(dinv * (aggp_ref[0] + zp_ref[0]) + b_ref[0], 0.0)
        e1 = jnp.maximum(dinv * (aggp_ref[1] + zp_ref[1]) + b_ref[1], 0.0)
        emb_ref[:, 0:half] = e0
        emb_ref[:, half:2 * half] = e1
        inv_n = 1.0 / float(n)
        p0 = jnp.sum(e0, axis=0, keepdims=True) * inv_n
        p1 = jnp.sum(e1, axis=0, keepdims=True) * inv_n
        part = _mm(p0, wfc_ref[0]) + _mm(p1, wfc_ref[1])

        @pl.when(i == 0)
        def _():
            logits_ref[...] = bfc_ref[...]

        logits_ref[...] += part

    return pl.pallas_call(
        body,
        grid=(n // RBLK,),
        in_specs=[
            pl.BlockSpec((NC, RBLK, half), lambda i: (0, i, 0)),
            pl.BlockSpec((NC, RBLK, half), lambda i: (0, i, 0)),
            pl.BlockSpec((RBLK, 1), lambda i: (i, 0)),
            pl.BlockSpec((NC, 1, half), lambda i: (0, 0, 0)),
            pl.BlockSpec((NC, half, out_d), lambda i: (0, 0, 0)),
            pl.BlockSpec((1, out_d), lambda i: (0, 0)),
        ],
        out_specs=[
            pl.BlockSpec((RBLK, 2 * half), lambda i: (i, 0)),
            pl.BlockSpec((1, out_d), lambda i: (0, 0)),
        ],
        out_shape=[jax.ShapeDtypeStruct((n, 2 * half), jnp.float32),
                   jax.ShapeDtypeStruct((1, out_d), jnp.float32)],
    )(aggp, zp, dinv, b_split, wfc_split, bfc)


def kernel(x, edge_index, batch, W1, b1, W2, b2, W3, b3, Wfc, bfc):
    n, d = x.shape
    e = edge_index.shape[1]
    out_d = Wfc.shape[1]
    half = d // 2

    grain = 2 * NC * NS * CHUNK
    e2 = ((e + grain - 1) // grain) * grain
    nch = e2 // (NS * CHUNK)
    nchw = e2 // (NC * NS * CHUNK)
    nacc = ((n + 1 + NS * 8 - 1) // (NS * 8)) * (NS * 8)
    rowz = nacc // NS

    src = edge_index[0]
    dst = edge_index[1]
    pad = e2 - e
    src_p = jnp.concatenate([src, jnp.zeros((pad,), jnp.int32)])
    dst_p = jnp.concatenate([dst, jnp.full((pad,), n, jnp.int32)])
    src_idx = jnp.stack([src_p, src_p + n]).reshape(NC, NS, nch, CHUNK)
    dst_main = dst_p.reshape(NS, nch, CHUNK)
    dst_deg = dst_p.reshape(NC * NS, nchw, CHUNK)
    ones_in = jnp.ones((CHUNK, half), jnp.float32)
    zrows128 = jnp.zeros((rowz, half), jnp.float32)

    degp = _sc_degree(dst_deg, ones_in, zrows128, n, nacc, nchw, half)

    w1s = W1.reshape(NC, half, d)
    w2s = W2.reshape(NC, half, d)
    w3s = W3.reshape(NC, half, d)
    wfcs = Wfc.reshape(NC, half, out_d)
    b1s = b1.reshape(NC, 1, half)
    b2s = b2.reshape(NC, 1, half)
    b3s = b3.reshape(NC, 1, half)
    bfc2 = bfc.reshape(1, out_d)

    z1, dinv = _tc_first(x, w1s, degp, n, half)
    agg1 = _sc_aggregate(z1.reshape(NC * n, half), src_idx, dst_main,
                         zrows128, n, half, nacc, nch
                         ).reshape(NC, nacc, half)[:, :n, :]
    z2 = _tc_mid(agg1, z1, dinv, b1s, w2s, n, half)
    agg2 = _sc_aggregate(z2.reshape(NC * n, half), src_idx, dst_main,
                         zrows128, n, half, nacc, nch
                         ).reshape(NC, nacc, half)[:, :n, :]
    z3 = _tc_mid(agg2, z2, dinv, b2s, w3s, n, half)
    agg3 = _sc_aggregate(z3.reshape(NC * n, half), src_idx, dst_main,
                         zrows128, n, half, nacc, nch
                         ).reshape(NC, nacc, half)[:, :n, :]
    emb, logits = _tc_final(agg3, z3, dinv, b3s, wfcs, bfc2, n, half, out_d)
    return (logits, emb)


# speedup vs baseline: 1.0936x; 1.0252x over previous
import functools

import jax
import jax.numpy as jnp
from jax import lax
from jax.experimental import pallas as pl
from jax.experimental.pallas import tpu as pltpu
from jax.experimental.pallas import tpu_sc as plsc

NC = 2
NS = 16
LANES = 16
CHUNK = 128


def _sc_degree(dst_deg, ones_in, zrows, n, nacc, nchw, width):
    rowz = nacc // NS
    mesh = plsc.VectorSubcoreMesh(core_axis_name="c", subcore_axis_name="s")

    @functools.partial(
        pl.kernel,
        mesh=mesh,
        out_type=jax.ShapeDtypeStruct((NC, nacc, width), jnp.float32),
        scratch_types=[
            pltpu.VMEM((nchw, CHUNK), jnp.int32),
            pltpu.VMEM((CHUNK, width), jnp.float32),
            pltpu.VMEM_SHARED((nacc, width), jnp.float32),
        ],
    )
    def degk(dst_hbm, ones_hbm, zrows_hbm, out_hbm, dstv, onesv, acc):
        c = lax.axis_index("c")
        s = lax.axis_index("s")
        w = s * NC + c
        pltpu.sync_copy(zrows_hbm, acc.at[pl.ds(s * rowz, rowz)])
        pltpu.sync_copy(dst_hbm.at[w], dstv)
        pltpu.sync_copy(ones_hbm, onesv)
        plsc.subcore_barrier()

        def body(j, carry):
            pltpu.sync_copy(onesv, acc.at[dstv.at[j]], add=True)
            return carry

        lax.fori_loop(0, nchw, body, 0)
        plsc.subcore_barrier()
        pltpu.sync_copy(acc.at[pl.ds(s * rowz, rowz)],
                        out_hbm.at[c, pl.ds(s * rowz, rowz)])

    return degk(dst_deg, ones_in, zrows)


def _sc_aggregate(z_flat, src_idx, dst_main, zrows, n, half, nacc, nch):
    rowz = nacc // NS
    mesh = plsc.VectorSubcoreMesh(core_axis_name="c", subcore_axis_name="s")

    nch2 = nch // 2

    @functools.partial(
        pl.kernel,
        mesh=mesh,
        out_type=jax.ShapeDtypeStruct((NC * nacc, half), jnp.float32),
        scratch_types=[
            pltpu.VMEM((nch2, CHUNK), jnp.int32),
            pltpu.VMEM((nch2, CHUNK), jnp.int32),
            pltpu.VMEM((CHUNK, half), jnp.float32),
            pltpu.VMEM((CHUNK, half), jnp.float32),
            pltpu.VMEM_SHARED((nacc, half), jnp.float32),
            pltpu.SemaphoreType.DMA,
            pltpu.SemaphoreType.DMA,
            pltpu.SemaphoreType.DMA,
            pltpu.SemaphoreType.DMA,
        ],
    )
    def aggk(z_hbm, src_hbm, dst_hbm, zrows_hbm, out_hbm,
             srcv, dstv, gbuf0, gbuf1, acc, g0, g1, s0, s1):
        c = lax.axis_index("c")
        s = lax.axis_index("s")
        pltpu.sync_copy(zrows_hbm, acc.at[pl.ds(s * rowz, rowz)])
        plsc.subcore_barrier()

        for p in range(2):
            pltpu.sync_copy(src_hbm.at[c, s, pl.ds(p * nch2, nch2)], srcv)
            pltpu.sync_copy(dst_hbm.at[s, pl.ds(p * nch2, nch2)], dstv)
            pltpu.async_copy(z_hbm.at[srcv.at[0]], gbuf0, g0)

            def body(i, carry):
                j = 2 * i
                pltpu.async_copy(z_hbm.at[srcv.at[j + 1]], gbuf1, g1)
                pltpu.make_async_copy(z_hbm.at[srcv.at[j]], gbuf0, g0).wait()
                pltpu.sync_copy(gbuf0, acc.at[dstv.at[j]], add=True)
                jn = jnp.minimum(j + 2, nch2 - 1)
                pltpu.async_copy(z_hbm.at[srcv.at[jn]], gbuf0, g0)
                pltpu.make_async_copy(z_hbm.at[srcv.at[j + 1]], gbuf1, g1).wait()
                pltpu.sync_copy(gbuf1, acc.at[dstv.at[j + 1]], add=True)
                return carry

            lax.fori_loop(0, nch2 // 2, body, 0)
            pltpu.make_async_copy(z_hbm.at[srcv.at[0]], gbuf0, g0).wait()
        plsc.subcore_barrier()
        pltpu.sync_copy(acc.at[pl.ds(s * rowz, rowz)],
                        out_hbm.at[pl.ds(c * nacc + s * rowz, rowz)])

    return aggk(z_flat, src_idx, dst_main, zrows)


RBLK = 2000


def _mm(a, b):
    return jnp.dot(a, b, preferred_element_type=jnp.float32,
                   precision=lax.Precision.HIGHEST)


def _tc_first(x, w_split, degp, n, half):
    def body(x_ref, w_ref, degp_ref, z_ref, dinv_ref):
        deg = degp_ref[0, :, 0:1] + degp_ref[1, :, 0:1] + 1.0
        dinv = lax.rsqrt(deg)
        dinv_ref[...] = dinv
        xv = x_ref[...]
        for c in range(NC):
            y = _mm(xv[:, :half], w_ref[0, :, c * half:(c + 1) * half])
            y += _mm(xv[:, half:], w_ref[1, :, c * half:(c + 1) * half])
            z_ref[c] = dinv * y

    return pl.pallas_call(
        body,
        grid=(n // RBLK,),
        in_specs=[
            pl.BlockSpec((RBLK, 2 * half), lambda i: (i, 0)),
            pl.BlockSpec((NC, half, 2 * half), lambda i: (0, 0, 0)),
            pl.BlockSpec((NC, RBLK, half), lambda i: (0, i, 0)),
        ],
        out_specs=[
            pl.BlockSpec((NC, RBLK, half), lambda i: (0, i, 0)),
            pl.BlockSpec((RBLK, 1), lambda i: (i, 0)),
        ],
        out_shape=[jax.ShapeDtypeStruct((NC, n, half), jnp.float32),
                   jax.ShapeDtypeStruct((n, 1), jnp.float32)],
    )(x, w_split, degp)


def _tc_mid(aggp, zp, dinv, b_split, w_split, n, half):
    def body(aggp_ref, zp_ref, dinv_ref, b_ref, w_ref, zout_ref):
        dinv = dinv_ref[...]
        h0 = jnp.maximum(dinv * (aggp_ref[0] + zp_ref[0]) + b_ref[0], 0.0)
        h1 = jnp.maximum(dinv * (aggp_ref[1] + zp_ref[1]) + b_ref[1], 0.0)
        for c in range(NC):
            y = _mm(h0, w_ref[0, :, c * half:(c + 1) * half])
            y += _mm(h1, w_ref[1, :, c * half:(c + 1) * half])
            zout_ref[c] = dinv * y

    return pl.pallas_call(
        body,
        grid=(n // RBLK,),
        in_specs=[
            pl.BlockSpec((NC, RBLK, half), lambda i: (0, i, 0)),
            pl.BlockSpec((NC, RBLK, half), lambda i: (0, i, 0)),
            pl.BlockSpec((RBLK, 1), lambda i: (i, 0)),
            pl.BlockSpec((NC, 1, half), lambda i: (0, 0, 0)),
            pl.BlockSpec((NC, half, 2 * half), lambda i: (0, 0, 0)),
        ],
        out_specs=pl.BlockSpec((NC, RBLK, half), lambda i: (0, i, 0)),
        out_shape=jax.ShapeDtypeStruct((NC, n, half), jnp.float32),
    )(aggp, zp, dinv, b_split, w_split)


def _tc_final(aggp, zp, dinv, b_split, wfc_split, bfc, n, half, out_d):
    def body(aggp_ref, zp_ref, dinv_ref, b_ref, wfc_ref, bfc_ref,
             emb_ref, logits_ref):
        i = pl.program_id(0)
        dinv = dinv_ref[...]
        e0 = jnp.maximum(dinv * (aggp_ref[0] + zp_ref[0]) + b_ref[0], 0.0)
        e1 = jnp.maximum(dinv * (aggp_ref[1] + zp_ref[1]) + b_ref[1], 0.0)
        emb_ref[:, 0:half] = e0
        emb_ref[:, half:2 * half] = e1
        inv_n = 1.0 / float(n)
        p0 = jnp.sum(e0, axis=0, keepdims=True) * inv_n
        p1 = jnp.sum(e1, axis=0, keepdims=True) * inv_n
        part = _mm(p0, wfc_ref[0]) + _mm(p1, wfc_ref[1])

        @pl.when(i == 0)
        def _():
            logits_ref[...] = bfc_ref[...]

        logits_ref[...] += part

    return pl.pallas_call(
        body,
        grid=(n // RBLK,),
        in_specs=[
            pl.BlockSpec((NC, RBLK, half), lambda i: (0, i, 0)),
            pl.BlockSpec((NC, RBLK, half), lambda i: (0, i, 0)),
            pl.BlockSpec((RBLK, 1), lambda i: (i, 0)),
            pl.BlockSpec((NC, 1, half), lambda i: (0, 0, 0)),
            pl.BlockSpec((NC, half, out_d), lambda i: (0, 0, 0)),
            pl.BlockSpec((1, out_d), lambda i: (0, 0)),
        ],
        out_specs=[
            pl.BlockSpec((RBLK, 2 * half), lambda i: (i, 0)),
            pl.BlockSpec((1, out_d), lambda i: (0, 0)),
        ],
        out_shape=[jax.ShapeDtypeStruct((n, 2 * half), jnp.float32),
                   jax.ShapeDtypeStruct((1, out_d), jnp.float32)],
    )(aggp, zp, dinv, b_split, wfc_split, bfc)


def kernel(x, edge_index, batch, W1, b1, W2, b2, W3, b3, Wfc, bfc):
    n, d = x.shape
    e = edge_index.shape[1]
    out_d = Wfc.shape[1]
    half = d // 2

    grain = 2 * NC * NS * CHUNK
    e2 = ((e + grain - 1) // grain) * grain
    nch = e2 // (NS * CHUNK)
    nchw = e2 // (NC * NS * CHUNK)
    nacc = ((n + 1 + NS * 8 - 1) // (NS * 8)) * (NS * 8)
    rowz = nacc // NS

    src = edge_index[0]
    dst = edge_index[1]
    pad = e2 - e
    src_p = jnp.concatenate([src, jnp.zeros((pad,), jnp.int32)])
    dst_p = jnp.concatenate([dst, jnp.full((pad,), n, jnp.int32)])
    src_idx = jnp.stack([src_p, src_p + n]).reshape(NC, NS, nch, CHUNK)
    dst_main = dst_p.reshape(NS, nch, CHUNK)
    dst_deg = dst_p.reshape(NC * NS, nchw, CHUNK)
    ones_in = jnp.ones((CHUNK, half), jnp.float32)
    zrows128 = jnp.zeros((rowz, half), jnp.float32)

    degp = _sc_degree(dst_deg, ones_in, zrows128, n, nacc, nchw, half)

    w1s = W1.reshape(NC, half, d)
    w2s = W2.reshape(NC, half, d)
    w3s = W3.reshape(NC, half, d)
    wfcs = Wfc.reshape(NC, half, out_d)
    b1s = b1.reshape(NC, 1, half)
    b2s = b2.reshape(NC, 1, half)
    b3s = b3.reshape(NC, 1, half)
    bfc2 = bfc.reshape(1, out_d)

    z1, dinv = _tc_first(x, w1s, degp, n, half)
    agg1 = _sc_aggregate(z1.reshape(NC * n, half), src_idx, dst_main,
                         zrows128, n, half, nacc, nch
                         ).reshape(NC, nacc, half)[:, :n, :]
    z2 = _tc_mid(agg1, z1, dinv, b1s, w2s, n, half)
    agg2 = _sc_aggregate(z2.reshape(NC * n, half), src_idx, dst_main,
                         zrows128, n, half, nacc, nch
                         ).reshape(NC, nacc, half)[:, :n, :]
    z3 = _tc_mid(agg2, z2, dinv, b2s, w3s, n, half)
    agg3 = _sc_aggregate(z3.reshape(NC * n, half), src_idx, dst_main,
                         zrows128, n, half, nacc, nch
                         ).reshape(NC, nacc, half)[:, :n, :]
    emb, logits = _tc_final(agg3, z3, dinv, b3s, wfcs, bfc2, n, half, out_d)
    return (logits, emb)
